# Initial kernel scaffold; baseline (speedup 1.0000x reference)
import jax
import jax.numpy as jnp
from jax.experimental import pallas as pl


def kernel(x, edge_index, W1, b1, W2, b2):
    raise NotImplementedError("write your pallas kernel here")


# speedup vs baseline: 21.3137x; 21.3137x over previous
import functools

import jax
import jax.numpy as jnp
from jax import lax
from jax.experimental import pallas as pl
from jax.experimental.pallas import tpu as pltpu
from jax.experimental.pallas import tpu_sc as plsc

_NC = 2
_NS = 16
_NW = _NC * _NS
_CH = 128

_N = 10000
_NT = 10112
_PER = _NT // _NS


def _sc_mesh():
    return plsc.VectorSubcoreMesh(
        core_axis_name="c", subcore_axis_name="s",
        num_cores=_NC, num_subcores=_NS)


def _hist_body(cpw, col_hbm, out_hbm, col_v, ones_v, slice_v, sem, deg_sh):
    cid = lax.axis_index("c")
    sid = lax.axis_index("s")
    w = cid * _NS + sid

    def zb(j, _):
        slice_v[pl.ds(j * 16, 16)] = jnp.zeros((16,), jnp.float32)
        return 0
    lax.fori_loop(0, _PER // 16, zb, 0)
    pltpu.sync_copy(slice_v, deg_sh.at[pl.ds(sid * _PER, _PER)])
    for j in range(_CH // 16):
        ones_v[pl.ds(j * 16, 16)] = jnp.ones((16,), jnp.float32)
    plsc.subcore_barrier()

    def body(i, _):
        base = (w * cpw + i) * _CH
        pltpu.sync_copy(col_hbm.at[pl.ds(base, _CH)], col_v)
        pltpu.sync_copy(ones_v, deg_sh.at[col_v], add=True)
        return 0
    lax.fori_loop(0, cpw, body, 0)
    plsc.subcore_barrier()

    pltpu.sync_copy(deg_sh.at[pl.ds(sid * _PER, _PER)], slice_v)
    pltpu.sync_copy(slice_v, out_hbm.at[pl.ds(cid * _NT + sid * _PER, _PER)])


def _make_hist(cpw):
    return functools.partial(
        pl.kernel, mesh=_sc_mesh(),
        out_type=jax.ShapeDtypeStruct((_NC * _NT,), jnp.float32),
        scratch_types=[
            pltpu.VMEM((_CH,), jnp.int32),
            pltpu.VMEM((_CH,), jnp.float32),
            pltpu.VMEM((_PER,), jnp.float32),
            pltpu.SemaphoreType.DMA,
            pltpu.VMEM_SHARED((_NT,), jnp.float32),
        ],
    )(functools.partial(_hist_body, cpw))


def _agg_body(cpw, src_hbm, row_hbm, col_hbm, out_hbm,
              row_v, col_v, msg_v, buf_v, sem, tbl_sh):
    cid = lax.axis_index("c")
    sid = lax.axis_index("s")
    w = cid * _NS + sid

    def zb(j, _):
        buf_v[j, :] = jnp.zeros((16,), jnp.float32)
        return 0
    lax.fori_loop(0, _PER, zb, 0)
    pltpu.sync_copy(buf_v, tbl_sh.at[pl.ds(sid * _PER, _PER)])
    plsc.subcore_barrier()

    def body(i, _):
        base = (w * cpw + i) * _CH
        pltpu.sync_copy(row_hbm.at[pl.ds(base, _CH)], row_v)
        pltpu.sync_copy(col_hbm.at[pl.ds(base, _CH)], col_v)
        pltpu.async_copy(src_hbm.at[row_v], msg_v, sem).wait()
        pltpu.sync_copy(msg_v, tbl_sh.at[col_v], add=True)
        return 0
    lax.fori_loop(0, cpw, body, 0)
    plsc.subcore_barrier()

    pltpu.sync_copy(tbl_sh.at[pl.ds(sid * _PER, _PER)], buf_v)
    pltpu.sync_copy(buf_v, out_hbm.at[pl.ds(cid * _NT + sid * _PER, _PER)])


def _make_agg(cpw):
    return functools.partial(
        pl.kernel, mesh=_sc_mesh(),
        compiler_params=pltpu.CompilerParams(use_tc_tiling_on_sc=False),
        out_type=jax.ShapeDtypeStruct((_NC * _NT, 16), jnp.float32),
        scratch_types=[
            pltpu.VMEM((_CH,), jnp.int32),
            pltpu.VMEM((_CH,), jnp.int32),
            pltpu.VMEM((_CH, 16), jnp.float32),
            pltpu.VMEM((_PER, 16), jnp.float32),
            pltpu.SemaphoreType.DMA,
            pltpu.VMEM_SHARED((_NT, 16), jnp.float32),
        ],
    )(functools.partial(_agg_body, cpw))


_BN = 2000


def _mm1_body(x_ref, w_ref, o_ref):
    o_ref[...] = jnp.dot(x_ref[...], w_ref[...],
                         preferred_element_type=jnp.float32)


def _dinv_body(degp_ref, o_ref):
    deg = degp_ref[0, :] + degp_ref[1, :] + 1.0
    o_ref[...] = lax.rsqrt(deg)[:, None]


def _scale_body(dinv_ref, h1_ref, o_ref):
    o_ref[...] = h1_ref[...] * dinv_ref[...]


def _post1_body(dinv_ref, p1_ref, h1s_ref, b1_ref, o_ref):
    dinv = dinv_ref[...]
    agg = (p1_ref[0] + p1_ref[1] + h1s_ref[...]) * dinv
    z = jnp.maximum(agg + b1_ref[...], 0.0)
    o_ref[...] = z * dinv


def _post2_body(dinv_ref, p2_ref, zs_ref, w2_ref, b2_ref, o_ref):
    dinv = dinv_ref[...]
    agg = (p2_ref[0] + p2_ref[1] + zs_ref[...]) * dinv
    o_ref[...] = jnp.dot(agg, w2_ref[...],
                         preferred_element_type=jnp.float32) + b2_ref[...]


def kernel(x, edge_index, W1, b1, W2, b2):
    n, d_in = x.shape
    hid = W1.shape[1]
    d_out = W2.shape[1]
    e = edge_index.shape[1]
    assert n == _N and hid == 16

    quant = _NW * _CH
    ep = ((e + quant - 1) // quant) * quant
    cpw = ep // quant
    pad = ep - e
    rowp = jnp.concatenate(
        [edge_index[0], jnp.zeros((pad,), jnp.int32)])
    colp = jnp.concatenate(
        [edge_index[1], jnp.full((pad,), n, jnp.int32)])

    grid = n // _BN
    h1 = pl.pallas_call(
        _mm1_body,
        grid=(grid,),
        in_specs=[pl.BlockSpec((_BN, d_in), lambda i: (i, 0)),
                  pl.BlockSpec((d_in, hid), lambda i: (0, 0))],
        out_specs=pl.BlockSpec((_BN, hid), lambda i: (i, 0)),
        out_shape=jax.ShapeDtypeStruct((n, hid), jnp.float32),
    )(x, W1)

    degp = _make_hist(cpw)(colp).reshape(_NC, _NT)

    dinv = pl.pallas_call(
        _dinv_body,
        grid=(1,),
        in_specs=[pl.BlockSpec((_NC, _NT), lambda i: (0, 0))],
        out_specs=pl.BlockSpec((_NT, 1), lambda i: (0, 0)),
        out_shape=jax.ShapeDtypeStruct((_NT, 1), jnp.float32),
    )(degp)[:n]

    h1s = pl.pallas_call(
        _scale_body,
        grid=(grid,),
        in_specs=[pl.BlockSpec((_BN, 1), lambda i: (i, 0)),
                  pl.BlockSpec((_BN, hid), lambda i: (i, 0))],
        out_specs=pl.BlockSpec((_BN, hid), lambda i: (i, 0)),
        out_shape=jax.ShapeDtypeStruct((n, hid), jnp.float32),
    )(dinv, h1)

    agg = _make_agg(cpw)
    p1 = agg(h1s, rowp, colp).reshape(_NC, _NT, hid)

    b1r = b1.reshape(1, hid)
    zs = pl.pallas_call(
        _post1_body,
        grid=(grid,),
        in_specs=[pl.BlockSpec((_BN, 1), lambda i: (i, 0)),
                  pl.BlockSpec((_NC, _BN, hid), lambda i: (0, i, 0)),
                  pl.BlockSpec((_BN, hid), lambda i: (i, 0)),
                  pl.BlockSpec((1, hid), lambda i: (0, 0))],
        out_specs=pl.BlockSpec((_BN, hid), lambda i: (i, 0)),
        out_shape=jax.ShapeDtypeStruct((n, hid), jnp.float32),
    )(dinv, p1, h1s, b1r)

    p2 = agg(zs, rowp, colp).reshape(_NC, _NT, hid)

    b2r = b2.reshape(1, d_out)
    out = pl.pallas_call(
        _post2_body,
        grid=(grid,),
        in_specs=[pl.BlockSpec((_BN, 1), lambda i: (i, 0)),
                  pl.BlockSpec((_NC, _BN, hid), lambda i: (0, i, 0)),
                  pl.BlockSpec((_BN, hid), lambda i: (i, 0)),
                  pl.BlockSpec((hid, d_out), lambda i: (0, 0)),
                  pl.BlockSpec((1, d_out), lambda i: (0, 0))],
        out_specs=pl.BlockSpec((_BN, d_out), lambda i: (i, 0)),
        out_shape=jax.ShapeDtypeStruct((n, d_out), jnp.float32),
    )(dinv, p2, zs, W2, b2r)

    return out


# speedup vs baseline: 49.6130x; 2.3277x over previous
import functools

import jax
import jax.numpy as jnp
from jax import lax
from jax.experimental import pallas as pl
from jax.experimental.pallas import tpu as pltpu
from jax.experimental.pallas import tpu_sc as plsc

_NC = 2
_NS = 16
_NW = _NC * _NS
_CH = 128

_N = 10000
_NT = 10112
_PER = _NT // _NS


def _sc_mesh():
    return plsc.VectorSubcoreMesh(
        core_axis_name="c", subcore_axis_name="s",
        num_cores=_NC, num_subcores=_NS)


_HK = 8


def _hist_body(cpw, col_hbm, out_hbm, col_all, ones_v, slice_v, sem, deg_sh):
    cid = lax.axis_index("c")
    sid = lax.axis_index("s")
    w = cid * _NS + sid

    def zb(j, _):
        slice_v[pl.ds(j * 16, 16)] = jnp.zeros((16,), jnp.float32)
        return 0
    lax.fori_loop(0, _PER // 16, zb, 0)
    pltpu.sync_copy(slice_v, deg_sh.at[pl.ds(sid * _PER, _PER)])
    for j in range(_CH // 16):
        ones_v[pl.ds(j * 16, 16)] = jnp.ones((16,), jnp.float32)
    pltpu.sync_copy(col_hbm.at[pl.ds(w * cpw, cpw)], col_all)
    plsc.subcore_barrier()

    def body(i, _):
        pltpu.sync_copy(ones_v, deg_sh.at[col_all.at[i]], add=True)
        return 0
    lax.fori_loop(0, cpw, body, 0)
    plsc.subcore_barrier()

    pltpu.sync_copy(deg_sh.at[pl.ds(sid * _PER, _PER)], slice_v)
    pltpu.sync_copy(slice_v, out_hbm.at[pl.ds(cid * _NT + sid * _PER, _PER)])


def _make_hist(cpw):
    return functools.partial(
        pl.kernel, mesh=_sc_mesh(),
        compiler_params=pltpu.CompilerParams(use_tc_tiling_on_sc=False),
        out_type=jax.ShapeDtypeStruct((_NC * _NT,), jnp.float32),
        scratch_types=[
            pltpu.VMEM((cpw, _CH), jnp.int32),
            pltpu.VMEM((_CH,), jnp.float32),
            pltpu.VMEM((_PER,), jnp.float32),
            pltpu.SemaphoreType.DMA,
            pltpu.VMEM_SHARED((_NT,), jnp.float32),
        ],
    )(functools.partial(_hist_body, cpw))


def _agg_body(cpw, src_hbm, row_hbm, col_hbm, out_hbm,
              row_all, col_all, msg_a, msg_b, buf_v, gsa, gsb, tbl_sh):
    cid = lax.axis_index("c")
    sid = lax.axis_index("s")
    w = cid * _NS + sid

    def zb(j, _):
        buf_v[j, :] = jnp.zeros((16,), jnp.float32)
        return 0
    lax.fori_loop(0, _PER, zb, 0)
    pltpu.sync_copy(buf_v, tbl_sh.at[pl.ds(sid * _PER, _PER)])
    pltpu.sync_copy(row_hbm.at[pl.ds(w * cpw, cpw)], row_all)
    pltpu.sync_copy(col_hbm.at[pl.ds(w * cpw, cpw)], col_all)
    plsc.subcore_barrier()

    npair = cpw // 2
    pltpu.async_copy(src_hbm.at[row_all.at[0]], msg_a, gsa)

    def body(i, _):
        j0 = 2 * i
        j1 = j0 + 1
        pltpu.async_copy(src_hbm.at[row_all.at[j1]], msg_b, gsb)
        pltpu.make_async_copy(src_hbm.at[row_all.at[j0]], msg_a, gsa).wait()
        pltpu.sync_copy(msg_a, tbl_sh.at[col_all.at[j0]], add=True)

        @pl.when(i < npair - 1)
        def _prefetch():
            pltpu.async_copy(src_hbm.at[row_all.at[j0 + 2]], msg_a, gsa)

        pltpu.make_async_copy(src_hbm.at[row_all.at[j1]], msg_b, gsb).wait()
        pltpu.sync_copy(msg_b, tbl_sh.at[col_all.at[j1]], add=True)
        return 0
    lax.fori_loop(0, npair, body, 0)
    plsc.subcore_barrier()

    pltpu.sync_copy(tbl_sh.at[pl.ds(sid * _PER, _PER)], buf_v)
    pltpu.sync_copy(buf_v, out_hbm.at[pl.ds(cid * _NT + sid * _PER, _PER)])


def _make_agg(cpw):
    return functools.partial(
        pl.kernel, mesh=_sc_mesh(),
        compiler_params=pltpu.CompilerParams(use_tc_tiling_on_sc=False),
        out_type=jax.ShapeDtypeStruct((_NC * _NT, 16), jnp.float32),
        scratch_types=[
            pltpu.VMEM((cpw, _CH), jnp.int32),
            pltpu.VMEM((cpw, _CH), jnp.int32),
            pltpu.VMEM((_CH, 16), jnp.float32),
            pltpu.VMEM((_CH, 16), jnp.float32),
            pltpu.VMEM((_PER, 16), jnp.float32),
            pltpu.SemaphoreType.DMA,
            pltpu.SemaphoreType.DMA,
            pltpu.VMEM_SHARED((_NT, 16), jnp.float32),
        ],
    )(functools.partial(_agg_body, cpw))


_BN = 2000


def _mm1_body(x_ref, w_ref, o_ref):
    o_ref[...] = jnp.dot(x_ref[...], w_ref[...],
                         preferred_element_type=jnp.float32)


def _dinv_body(degp_ref, o_ref):
    deg = degp_ref[0, :] + degp_ref[1, :] + 1.0
    o_ref[...] = lax.rsqrt(deg)[:, None]


def _scale_body(dinv_ref, h1_ref, o_ref):
    o_ref[...] = h1_ref[...] * dinv_ref[...]


def _post1_body(dinv_ref, p1_ref, h1s_ref, b1_ref, o_ref):
    dinv = dinv_ref[...]
    agg = (p1_ref[0] + p1_ref[1] + h1s_ref[...]) * dinv
    z = jnp.maximum(agg + b1_ref[...], 0.0)
    o_ref[...] = z * dinv


def _post2_body(dinv_ref, p2_ref, zs_ref, w2_ref, b2_ref, o_ref):
    dinv = dinv_ref[...]
    agg = (p2_ref[0] + p2_ref[1] + zs_ref[...]) * dinv
    o_ref[...] = jnp.dot(agg, w2_ref[...],
                         preferred_element_type=jnp.float32) + b2_ref[...]


def kernel(x, edge_index, W1, b1, W2, b2):
    n, d_in = x.shape
    hid = W1.shape[1]
    d_out = W2.shape[1]
    e = edge_index.shape[1]
    assert n == _N and hid == 16

    quant = _NW * _CH * 8
    ep = ((e + quant - 1) // quant) * quant
    cpw = ep // (_NW * _CH)
    pad = ep - e
    parange = jnp.arange(pad, dtype=jnp.int32)
    rowp = jnp.concatenate(
        [edge_index[0], parange % n]).reshape(_NW * cpw, _CH)
    colp = jnp.concatenate(
        [edge_index[1], n + (parange % (_NT - n))]).reshape(_NW * cpw, _CH)

    grid = n // _BN
    h1 = pl.pallas_call(
        _mm1_body,
        grid=(grid,),
        in_specs=[pl.BlockSpec((_BN, d_in), lambda i: (i, 0)),
                  pl.BlockSpec((d_in, hid), lambda i: (0, 0))],
        out_specs=pl.BlockSpec((_BN, hid), lambda i: (i, 0)),
        out_shape=jax.ShapeDtypeStruct((n, hid), jnp.float32),
    )(x, W1)

    degp = _make_hist(cpw)(colp).reshape(_NC, _NT)

    dinv = pl.pallas_call(
        _dinv_body,
        grid=(1,),
        in_specs=[pl.BlockSpec((_NC, _NT), lambda i: (0, 0))],
        out_specs=pl.BlockSpec((_NT, 1), lambda i: (0, 0)),
        out_shape=jax.ShapeDtypeStruct((_NT, 1), jnp.float32),
    )(degp)[:n]

    h1s = pl.pallas_call(
        _scale_body,
        grid=(grid,),
        in_specs=[pl.BlockSpec((_BN, 1), lambda i: (i, 0)),
                  pl.BlockSpec((_BN, hid), lambda i: (i, 0))],
        out_specs=pl.BlockSpec((_BN, hid), lambda i: (i, 0)),
        out_shape=jax.ShapeDtypeStruct((n, hid), jnp.float32),
    )(dinv, h1)

    agg = _make_agg(cpw)
    p1 = agg(h1s, rowp, colp).reshape(_NC, _NT, hid)

    b1r = b1.reshape(1, hid)
    zs = pl.pallas_call(
        _post1_body,
        grid=(grid,),
        in_specs=[pl.BlockSpec((_BN, 1), lambda i: (i, 0)),
                  pl.BlockSpec((_NC, _BN, hid), lambda i: (0, i, 0)),
                  pl.BlockSpec((_BN, hid), lambda i: (i, 0)),
                  pl.BlockSpec((1, hid), lambda i: (0, 0))],
        out_specs=pl.BlockSpec((_BN, hid), lambda i: (i, 0)),
        out_shape=jax.ShapeDtypeStruct((n, hid), jnp.float32),
    )(dinv, p1, h1s, b1r)

    p2 = agg(zs, rowp, colp).reshape(_NC, _NT, hid)

    b2r = b2.reshape(1, d_out)
    out = pl.pallas_call(
        _post2_body,
        grid=(grid,),
        in_specs=[pl.BlockSpec((_BN, 1), lambda i: (i, 0)),
                  pl.BlockSpec((_NC, _BN, hid), lambda i: (0, i, 0)),
                  pl.BlockSpec((_BN, hid), lambda i: (i, 0)),
                  pl.BlockSpec((hid, d_out), lambda i: (0, 0)),
                  pl.BlockSpec((1, d_out), lambda i: (0, 0))],
        out_specs=pl.BlockSpec((_BN, d_out), lambda i: (i, 0)),
        out_shape=jax.ShapeDtypeStruct((n, d_out), jnp.float32),
    )(dinv, p2, zs, W2, b2r)

    return out


# speedup vs baseline: 59.4111x; 1.1975x over previous
import functools

import jax
import jax.numpy as jnp
from jax import lax
from jax.experimental import pallas as pl
from jax.experimental.pallas import tpu as pltpu
from jax.experimental.pallas import tpu_sc as plsc

_NC = 2
_NS = 16
_NW = _NC * _NS
_CH = 128
_G = 20

_N = 10000
_NT = 10240
_PER = _NT // _NS


def _sc_mesh():
    return plsc.VectorSubcoreMesh(
        core_axis_name="c", subcore_axis_name="s",
        num_cores=_NC, num_subcores=_NS)


_HK = 8


def _hist_body(cpw, col_hbm, out_hbm, col_all, ones_v, slice_v, sem, deg_sh):
    cid = lax.axis_index("c")
    sid = lax.axis_index("s")
    w = cid * _NS + sid

    def zb(j, _):
        slice_v[pl.ds(j * 16, 16)] = jnp.zeros((16,), jnp.float32)
        return 0
    lax.fori_loop(0, _PER // 16, zb, 0)
    pltpu.sync_copy(slice_v, deg_sh.at[pl.ds(sid * _PER, _PER)])

    def ob(j, _):
        ones_v[pl.ds(j * 16, 16)] = jnp.ones((16,), jnp.float32)
        return 0
    lax.fori_loop(0, _CH // 16, ob, 0)
    pltpu.sync_copy(col_hbm.at[pl.ds(w * cpw, cpw)], col_all)
    plsc.subcore_barrier()

    def body(i, _):
        pltpu.sync_copy(ones_v, deg_sh.at[col_all.at[i]], add=True)
        return 0
    lax.fori_loop(0, cpw, body, 0)
    plsc.subcore_barrier()

    pltpu.sync_copy(deg_sh.at[pl.ds(sid * _PER, _PER)], slice_v)
    pltpu.sync_copy(slice_v, out_hbm.at[pl.ds(cid * _NT + sid * _PER, _PER)])


def _make_hist(cpw):
    return functools.partial(
        pl.kernel, mesh=_sc_mesh(),
        compiler_params=pltpu.CompilerParams(use_tc_tiling_on_sc=False),
        out_type=jax.ShapeDtypeStruct((_NC * _NT,), jnp.float32),
        scratch_types=[
            pltpu.VMEM((cpw, _CH), jnp.int32),
            pltpu.VMEM((_CH,), jnp.float32),
            pltpu.VMEM((_PER,), jnp.float32),
            pltpu.SemaphoreType.DMA,
            pltpu.VMEM_SHARED((_NT,), jnp.float32),
        ],
    )(functools.partial(_hist_body, cpw))


def _agg_body(cpw, src_hbm, row_hbm, col_hbm, out_hbm,
              row_all, col_all, m0, m1, m2, m3, bufa,
              g0, g1, g2, g3, tbl_sh):
    cid = lax.axis_index("c")
    sid = lax.axis_index("s")
    w = cid * _NS + sid

    def zb(j, _):
        bufa[j, :] = jnp.zeros((16,), jnp.float32)
        return 0
    lax.fori_loop(0, _PER, zb, 0)
    pltpu.sync_copy(bufa, tbl_sh.at[pl.ds(sid * _PER, _PER)])
    pltpu.sync_copy(row_hbm.at[pl.ds(w * cpw, cpw)], row_all)
    pltpu.sync_copy(col_hbm.at[pl.ds(w * cpw, cpw)], col_all)
    plsc.subcore_barrier()

    msgs = (m0, m1, m2, m3)
    gs = (g0, g1, g2, g3)

    def _gather(j, k):
        pltpu.async_copy(src_hbm.at[row_all.at[j]], msgs[k], gs[k])

    def _gwait(j, k):
        pltpu.make_async_copy(src_hbm.at[row_all.at[j]], msgs[k], gs[k]).wait()

    def _scat(j, k):
        pltpu.sync_copy(msgs[k], tbl_sh.at[col_all.at[j]], add=True)

    for k in range(4):
        _gather(k, k)

    def body(i, _):
        for kp in range(4):
            j = 4 * i + kp
            _gwait(j, kp)
            _scat(j, kp)
            _gather(j + 4, kp)
        return 0
    lax.fori_loop(0, (cpw - 4) // 4, body, 0)
    for kp in range(4):
        j = cpw - 4 + kp
        _gwait(j, kp)
        _scat(j, kp)
    plsc.subcore_barrier()

    pltpu.sync_copy(tbl_sh.at[pl.ds(sid * _PER, _PER)], bufa)
    pltpu.sync_copy(bufa, out_hbm.at[pl.ds(cid * _NT + sid * _PER, _PER)])


def _make_agg(cpw):
    return functools.partial(
        pl.kernel, mesh=_sc_mesh(),
        compiler_params=pltpu.CompilerParams(use_tc_tiling_on_sc=False),
        out_type=jax.ShapeDtypeStruct((_NC * _NT, 16), jnp.float32),
        scratch_types=[
            pltpu.VMEM((cpw, _CH), jnp.int32),
            pltpu.VMEM((cpw, _CH), jnp.int32),
            pltpu.VMEM((_CH, 16), jnp.float32),
            pltpu.VMEM((_CH, 16), jnp.float32),
            pltpu.VMEM((_CH, 16), jnp.float32),
            pltpu.VMEM((_CH, 16), jnp.float32),
            pltpu.VMEM((_PER, 16), jnp.float32),
            pltpu.SemaphoreType.DMA,
            pltpu.SemaphoreType.DMA,
            pltpu.SemaphoreType.DMA,
            pltpu.SemaphoreType.DMA,
            pltpu.VMEM_SHARED((_NT, 16), jnp.float32),
        ],
    )(functools.partial(_agg_body, cpw))


_BN = 2000


def _mm1_body(x_ref, w_ref, o_ref):
    o_ref[...] = jnp.dot(x_ref[...], w_ref[...],
                         preferred_element_type=jnp.float32)


def _dinv_body(degp_ref, o_ref):
    deg = degp_ref[0, :] + degp_ref[1, :] + 1.0
    o_ref[...] = lax.rsqrt(deg)[:, None]


def _scale_body(dinv_ref, h1_ref, o_ref):
    o_ref[...] = h1_ref[...] * dinv_ref[...]


def _post1_body(dinv_ref, p1_ref, h1s_ref, b1_ref, o_ref):
    dinv = dinv_ref[...]
    agg = (p1_ref[0] + p1_ref[1] + h1s_ref[...]) * dinv
    z = jnp.maximum(agg + b1_ref[...], 0.0)
    o_ref[...] = z * dinv


def _post2_body(dinv_ref, p2_ref, zs_ref, w2_ref, b2_ref, o_ref):
    dinv = dinv_ref[...]
    agg = (p2_ref[0] + p2_ref[1] + zs_ref[...]) * dinv
    o_ref[...] = jnp.dot(agg, w2_ref[...],
                         preferred_element_type=jnp.float32) + b2_ref[...]


def kernel(x, edge_index, W1, b1, W2, b2):
    n, d_in = x.shape
    hid = W1.shape[1]
    d_out = W2.shape[1]
    e = edge_index.shape[1]
    assert n == _N and hid == 16

    quant = _NW * _CH * 4
    ep = ((e + quant - 1) // quant) * quant
    cpw = ep // (_NW * _CH)
    pad = ep - e
    parange = jnp.arange(pad, dtype=jnp.int32)
    rowp = jnp.concatenate(
        [edge_index[0], parange % n]).reshape(_NW * cpw, _CH)
    colp = jnp.concatenate(
        [edge_index[1], n + (parange % (_NT - n))]).reshape(_NW * cpw, _CH)

    grid = n // _BN
    h1 = pl.pallas_call(
        _mm1_body,
        grid=(grid,),
        in_specs=[pl.BlockSpec((_BN, d_in), lambda i: (i, 0)),
                  pl.BlockSpec((d_in, hid), lambda i: (0, 0))],
        out_specs=pl.BlockSpec((_BN, hid), lambda i: (i, 0)),
        out_shape=jax.ShapeDtypeStruct((n, hid), jnp.float32),
    )(x, W1)

    degp = _make_hist(cpw)(colp).reshape(_NC, _NT)

    dinv = pl.pallas_call(
        _dinv_body,
        grid=(1,),
        in_specs=[pl.BlockSpec((_NC, _NT), lambda i: (0, 0))],
        out_specs=pl.BlockSpec((_NT, 1), lambda i: (0, 0)),
        out_shape=jax.ShapeDtypeStruct((_NT, 1), jnp.float32),
    )(degp)[:n]

    h1s = pl.pallas_call(
        _scale_body,
        grid=(grid,),
        in_specs=[pl.BlockSpec((_BN, 1), lambda i: (i, 0)),
                  pl.BlockSpec((_BN, hid), lambda i: (i, 0))],
        out_specs=pl.BlockSpec((_BN, hid), lambda i: (i, 0)),
        out_shape=jax.ShapeDtypeStruct((n, hid), jnp.float32),
    )(dinv, h1)

    agg = _make_agg(cpw)
    p1 = agg(h1s, rowp, colp).reshape(_NC, _NT, hid)

    b1r = b1.reshape(1, hid)
    zs = pl.pallas_call(
        _post1_body,
        grid=(grid,),
        in_specs=[pl.BlockSpec((_BN, 1), lambda i: (i, 0)),
                  pl.BlockSpec((_NC, _BN, hid), lambda i: (0, i, 0)),
                  pl.BlockSpec((_BN, hid), lambda i: (i, 0)),
                  pl.BlockSpec((1, hid), lambda i: (0, 0))],
        out_specs=pl.BlockSpec((_BN, hid), lambda i: (i, 0)),
        out_shape=jax.ShapeDtypeStruct((n, hid), jnp.float32),
    )(dinv, p1, h1s, b1r)

    p2 = agg(zs, rowp, colp).reshape(_NC, _NT, hid)

    b2r = b2.reshape(1, d_out)
    out = pl.pallas_call(
        _post2_body,
        grid=(grid,),
        in_specs=[pl.BlockSpec((_BN, 1), lambda i: (i, 0)),
                  pl.BlockSpec((_NC, _BN, hid), lambda i: (0, i, 0)),
                  pl.BlockSpec((_BN, hid), lambda i: (i, 0)),
                  pl.BlockSpec((hid, d_out), lambda i: (0, 0)),
                  pl.BlockSpec((1, d_out), lambda i: (0, 0))],
        out_specs=pl.BlockSpec((_BN, d_out), lambda i: (i, 0)),
        out_shape=jax.ShapeDtypeStruct((n, d_out), jnp.float32),
    )(dinv, p2, zs, W2, b2r)

    return out
